# baseline (speedup 1.0000x reference)
import jax
import jax.numpy as jnp
import numpy as np
from jax.experimental import pallas as pl
from jax.experimental.pallas import tpu as pltpu

_B, _H, _W = 4, 512, 512
_P = 16
_G = 32
_N = _G * _G
_D = 192
_NH = 3
_DH = _D // _NH
_DFF = 4 * _D
_NBLK = 2
_THR = 0.5
_SCALE = 1.0 / np.sqrt(_DH)

_BLOCK_KEYS = ('ln1_g', 'ln1_b', 'qkv_w', 'qkv_b', 'proj_w', 'proj_b',
               'ln2_g', 'ln2_b', 'fc1_w', 'fc1_b', 'fc2_w', 'fc2_b')


def _mm(a, b):
    return jnp.dot(a.astype(jnp.bfloat16), b.astype(jnp.bfloat16),
                   preferred_element_type=jnp.float32)


def _ln(x, g, b):
    mu = jnp.mean(x, -1, keepdims=True)
    v = jnp.mean((x - mu) ** 2, -1, keepdims=True)
    return (x - mu) * jax.lax.rsqrt(v + 1e-6) * g + b


def _attn(qkv, keep_t):
    outs = []
    for h in range(_NH):
        q = qkv[:, h * _DH:(h + 1) * _DH]
        k = qkv[:, _D + h * _DH:_D + (h + 1) * _DH]
        v = qkv[:, 2 * _D + h * _DH:2 * _D + (h + 1) * _DH]
        logits = jax.lax.dot_general(
            q.astype(jnp.bfloat16), k.astype(jnp.bfloat16),
            (((1,), (1,)), ((), ())),
            preferred_element_type=jnp.float32) * _SCALE
        if keep_t is not None:
            logits = jnp.where(keep_t, logits, -1e30)
        m = jnp.max(logits, axis=-1, keepdims=True)
        e = jnp.exp(logits - m)
        att = e / jnp.sum(e, axis=-1, keepdims=True)
        outs.append(_mm(att, v))
    return jnp.concatenate(outs, axis=-1)


def _tblock(x, w, keep_t=None):
    g1, b1, qw, qb, pw, pb, g2, b2, f1w, f1b, f2w, f2b = w
    h = _ln(x, g1, b1)
    qkv = _mm(h, qw) + qb
    o = _attn(qkv, keep_t)
    x = x + _mm(o, pw) + pb
    h2 = _ln(x, g2, b2)
    ff = jax.nn.gelu(_mm(h2, f1w) + f1b)
    x = x + _mm(ff, f2w) + f2b
    return x


def _body(patches_ref, conf_ref, *refs):
    nw = 7 + _NBLK * len(_BLOCK_KEYS)
    w = [r[...] for r in refs[:nw]]
    dtok_ref, cg_ref = refs[nw], refs[nw + 1]
    patch_w, patch_b = w[0], w[1]
    blocks = [w[2 + i * 12:2 + (i + 1) * 12] for i in range(_NBLK)]
    norm_g, norm_b, head_w, head_b, fill = w[2 + 12 * _NBLK:]

    conf = conf_ref[0, 0]
    rband = jnp.sum(conf.reshape(_G, _P, _W), axis=1)
    lane = jax.lax.broadcasted_iota(jnp.int32, (_W, _G), 0)
    col = jax.lax.broadcasted_iota(jnp.int32, (_W, _G), 1)
    blk_ones = (lane // _P == col).astype(jnp.float32)
    cg_grid = jnp.dot(rband, blk_ones, precision=jax.lax.Precision.HIGHEST,
                      preferred_element_type=jnp.float32) * (1.0 / (_P * _P))
    tok = jax.lax.broadcasted_iota(jnp.int32, (_N, _G), 0)
    j = jax.lax.broadcasted_iota(jnp.int32, (_N, _G), 1)
    oh_gy = (tok // _G == j).astype(jnp.float32)
    oh_gx = (tok % _G == j).astype(jnp.float32)
    per_gy = jnp.dot(oh_gy, cg_grid, precision=jax.lax.Precision.HIGHEST,
                     preferred_element_type=jnp.float32)
    cg = jnp.sum(per_gy * oh_gx, axis=-1, keepdims=True)
    cg_ref[0] = cg
    keep = cg < _THR
    keep_t = jax.lax.transpose(keep, (1, 0))

    x = _mm(patches_ref[0], patch_w) + patch_b

    att = _tblock(x, blocks[0], keep_t=keep_t)
    x = jnp.where(keep, att, fill)

    for bw in blocks:
        x = _tblock(x, bw)

    x = _ln(x, norm_g, norm_b)
    dtok_ref[0] = _mm(x, head_w) + head_b


def _full_spec(shape):
    nd = len(shape)
    return pl.BlockSpec(shape, lambda b: (0,) * nd)


def kernel(image, confidence_map, sgm_depth_prior, params):
    del sgm_depth_prior

    def w2d(a):
        return a.reshape(1, -1) if a.ndim == 1 else a

    patches = image.reshape(_B, 3, _G, _P, _G, _P).transpose(
        0, 2, 4, 1, 3, 5).reshape(_B, _N, _P * _P * 3)
    pw = params['patch_w'].reshape(_P, _P, 3, _D).transpose(
        2, 0, 1, 3).reshape(_P * _P * 3, _D)

    weights = [pw, w2d(params['patch_b'])]
    for bp in params['blocks']:
        weights.extend(w2d(bp[k]) for k in _BLOCK_KEYS)
    weights.extend([w2d(params['norm_g']), w2d(params['norm_b']),
                    w2d(params['head_w']), w2d(params['head_b']),
                    w2d(params['fill_token'])])

    in_specs = [
        pl.BlockSpec((1, _N, _P * _P * 3), lambda b: (b, 0, 0)),
        pl.BlockSpec((1, 1, _H, _W), lambda b: (b, 0, 0, 0)),
    ] + [_full_spec(wt.shape) for wt in weights]

    dtok, cg = pl.pallas_call(
        _body,
        grid=(_B,),
        in_specs=in_specs,
        out_specs=[
            pl.BlockSpec((1, _N, _P * _P), lambda b: (b, 0, 0)),
            pl.BlockSpec((1, _N, 1), lambda b: (b, 0, 0)),
        ],
        out_shape=[
            jax.ShapeDtypeStruct((_B, _N, _P * _P), jnp.float32),
            jax.ShapeDtypeStruct((_B, _N, 1), jnp.float32),
        ],
        compiler_params=pltpu.CompilerParams(
            dimension_semantics=("parallel",)),
    )(patches, confidence_map, *weights)

    depth = dtok.reshape(_B, _G, _G, _P, _P).transpose(
        0, 1, 3, 2, 4).reshape(_B, 1, _H, _W)
    cg_flat = cg.reshape(_B, _N)
    prune_ratio = jnp.mean((cg_flat >= _THR).astype(jnp.float32))
    return depth, prune_ratio, cg_flat.reshape(_B, _G, _G)


# speedup vs baseline: 1.6541x; 1.6541x over previous
import jax
import jax.numpy as jnp
import numpy as np
from jax.experimental import pallas as pl
from jax.experimental.pallas import tpu as pltpu

_B, _H, _W = 4, 512, 512
_P = 16
_G = 32
_N = _G * _G
_D = 192
_NH = 3
_DH = _D // _NH
_DFF = 4 * _D
_NBLK = 2
_THR = 0.5
_SCALE = 1.0 / np.sqrt(_DH)

_BLOCK_KEYS = ('ln1_g', 'ln1_b', 'qkv_w', 'qkv_b', 'proj_w', 'proj_b',
               'ln2_g', 'ln2_b', 'fc1_w', 'fc1_b', 'fc2_w', 'fc2_b')


def _mm(a, b):
    return jnp.dot(a.astype(jnp.bfloat16), b.astype(jnp.bfloat16),
                   preferred_element_type=jnp.float32)


def _ln(x, g, b):
    mu = jnp.mean(x, -1, keepdims=True)
    v = jnp.mean((x - mu) ** 2, -1, keepdims=True)
    return (x - mu) * jax.lax.rsqrt(v + 1e-6) * g + b


def _attn(qkv, keep_t):
    outs = []
    for h in range(_NH):
        q = qkv[:, h * _DH:(h + 1) * _DH]
        k = qkv[:, _D + h * _DH:_D + (h + 1) * _DH]
        v = qkv[:, 2 * _D + h * _DH:2 * _D + (h + 1) * _DH]
        logits = jax.lax.dot_general(
            q.astype(jnp.bfloat16), k.astype(jnp.bfloat16),
            (((1,), (1,)), ((), ())),
            preferred_element_type=jnp.float32) * _SCALE
        if keep_t is not None:
            logits = jnp.where(keep_t, logits, -1e30)
        m = jnp.max(logits, axis=-1, keepdims=True)
        e = jnp.exp(logits - m)
        att = e / jnp.sum(e, axis=-1, keepdims=True)
        outs.append(_mm(att, v))
    return jnp.concatenate(outs, axis=-1)


def _tblock(x, w, keep_t=None):
    g1, b1, qw, qb, pw, pb, g2, b2, f1w, f1b, f2w, f2b = w
    h = _ln(x, g1, b1)
    qkv = _mm(h, qw) + qb
    o = _attn(qkv, keep_t)
    x = x + _mm(o, pw) + pb
    h2 = _ln(x, g2, b2)
    ff = jax.nn.gelu(_mm(h2, f1w) + f1b)
    x = x + _mm(ff, f2w) + f2b
    return x


def _body(image_ref, conf_ref, *refs):
    nw = 7 + _NBLK * len(_BLOCK_KEYS)
    w = [r[...] for r in refs[:nw]]
    depth_ref, cg_ref = refs[nw], refs[nw + 1]
    st_s, dt_s = refs[nw + 2], refs[nw + 3]
    patch_w, patch_b = w[0], w[1]
    blocks = [w[2 + i * 12:2 + (i + 1) * 12] for i in range(_NBLK)]
    norm_g, norm_b, head_w, head_b, fill = w[2 + 12 * _NBLK:]

    conf = conf_ref[0, 0]
    rband = jnp.sum(conf.reshape(_G, _P, _W), axis=1)
    lane = jax.lax.broadcasted_iota(jnp.int32, (_W, _G), 0)
    col = jax.lax.broadcasted_iota(jnp.int32, (_W, _G), 1)
    blk_ones = (lane // _P == col).astype(jnp.float32)
    cg_grid = jnp.dot(rband, blk_ones, precision=jax.lax.Precision.HIGHEST,
                      preferred_element_type=jnp.float32) * (1.0 / (_P * _P))
    tok = jax.lax.broadcasted_iota(jnp.int32, (_N, _G), 0)
    j = jax.lax.broadcasted_iota(jnp.int32, (_N, _G), 1)
    oh_gy = (tok // _G == j).astype(jnp.float32)
    oh_gx = (tok % _G == j).astype(jnp.float32)
    per_gy = jnp.dot(oh_gy, cg_grid, precision=jax.lax.Precision.HIGHEST,
                     preferred_element_type=jnp.float32)
    cg = jnp.sum(per_gy * oh_gx, axis=-1, keepdims=True)
    cg_ref[0] = cg
    keep = cg < _THR
    keep_t = jax.lax.transpose(keep, (1, 0))

    bands = []
    for gy in range(_G):
        slab = image_ref[0, :, gy * _P:(gy + 1) * _P, :].reshape(
            3 * _P, _W)
        st_s[gy] = jax.lax.transpose(slab, (1, 0))
    for gy in range(_G):
        parts = [st_s[gy, px::_P, :] for px in range(_P)]
        bands.append(jnp.concatenate(parts, axis=1))
    patches = jnp.concatenate(bands, axis=0)
    x = _mm(patches, patch_w) + patch_b

    att = _tblock(x, blocks[0], keep_t=keep_t)
    x = jnp.where(keep, att, fill)

    for bw in blocks:
        x = _tblock(x, bw)

    x = _ln(x, norm_g, norm_b)
    dtok = _mm(x, head_w) + head_b

    for gy in range(_G):
        dg = dtok[gy * _G:(gy + 1) * _G, :]
        for px in range(_P):
            dt_s[gy, px::_P, :] = dg[:, px * _P:(px + 1) * _P]
    for gy in range(_G):
        band = jax.lax.transpose(dt_s[gy], (1, 0))
        depth_ref[0, 0, gy * _P:(gy + 1) * _P, :] = band


def _full_spec(shape):
    nd = len(shape)
    return pl.BlockSpec(shape, lambda b: (0,) * nd)


def kernel(image, confidence_map, sgm_depth_prior, params):
    del sgm_depth_prior

    def w2d(a):
        return a.reshape(1, -1) if a.ndim == 1 else a

    pw = params['patch_w'].reshape(_P, _P, 3, _D).transpose(
        1, 2, 0, 3).reshape(_P * _P * 3, _D)
    hw = params['head_w'].reshape(_D, _P, _P).transpose(
        0, 2, 1).reshape(_D, _P * _P)
    hb = params['head_b'].reshape(_P, _P).transpose(1, 0).reshape(1, _P * _P)

    weights = [pw, w2d(params['patch_b'])]
    for bp in params['blocks']:
        weights.extend(w2d(bp[k]) for k in _BLOCK_KEYS)
    weights.extend([w2d(params['norm_g']), w2d(params['norm_b']),
                    hw, hb, w2d(params['fill_token'])])

    in_specs = [
        pl.BlockSpec((1, 3, _H, _W), lambda b: (b, 0, 0, 0)),
        pl.BlockSpec((1, 1, _H, _W), lambda b: (b, 0, 0, 0)),
    ] + [_full_spec(wt.shape) for wt in weights]

    depth, cg = pl.pallas_call(
        _body,
        grid=(_B,),
        in_specs=in_specs,
        out_specs=[
            pl.BlockSpec((1, 1, _H, _W), lambda b: (b, 0, 0, 0)),
            pl.BlockSpec((1, _N, 1), lambda b: (b, 0, 0)),
        ],
        out_shape=[
            jax.ShapeDtypeStruct((_B, 1, _H, _W), jnp.float32),
            jax.ShapeDtypeStruct((_B, _N, 1), jnp.float32),
        ],
        scratch_shapes=[
            pltpu.VMEM((_G, _W, 48), jnp.float32),
            pltpu.VMEM((_G, _W, _P), jnp.float32),
        ],
        compiler_params=pltpu.CompilerParams(
            dimension_semantics=("parallel",)),
    )(image, confidence_map, *weights)

    cg_flat = cg.reshape(_B, _N)
    prune_ratio = jnp.mean((cg_flat >= _THR).astype(jnp.float32))
    return depth, prune_ratio, cg_flat.reshape(_B, _G, _G)


# speedup vs baseline: 2.0429x; 1.2350x over previous
import jax
import jax.numpy as jnp
import numpy as np
from jax.experimental import pallas as pl
from jax.experimental.pallas import tpu as pltpu

_B, _H, _W = 4, 512, 512
_P = 16
_G = 32
_N = _G * _G
_D = 192
_NH = 3
_DH = _D // _NH
_DFF = 4 * _D
_NBLK = 2
_THR = 0.5
_SCALE = 1.0 / np.sqrt(_DH)

_BLOCK_KEYS = ('ln1_g', 'ln1_b', 'qkv_w', 'qkv_b', 'proj_w', 'proj_b',
               'ln2_g', 'ln2_b', 'fc1_w', 'fc1_b', 'fc2_w', 'fc2_b')


def _mm(a, b):
    return jnp.dot(a.astype(jnp.bfloat16), b.astype(jnp.bfloat16),
                   preferred_element_type=jnp.float32)


def _ln(x, g, b):
    mu = jnp.mean(x, -1, keepdims=True)
    v = jnp.mean((x - mu) ** 2, -1, keepdims=True)
    return (x - mu) * jax.lax.rsqrt(v + 1e-6) * g + b


def _attn(qkv, keep_t):
    outs = []
    for h in range(_NH):
        q = qkv[:, h * _DH:(h + 1) * _DH]
        k = qkv[:, _D + h * _DH:_D + (h + 1) * _DH]
        v = qkv[:, 2 * _D + h * _DH:2 * _D + (h + 1) * _DH]
        logits = jax.lax.dot_general(
            q.astype(jnp.bfloat16), k.astype(jnp.bfloat16),
            (((1,), (1,)), ((), ())),
            preferred_element_type=jnp.float32) * _SCALE
        if keep_t is not None:
            logits = jnp.where(keep_t, logits, -1e30)
        e = jnp.exp(logits)
        att = e / jnp.sum(e, axis=-1, keepdims=True)
        outs.append(_mm(att, v))
    return jnp.concatenate(outs, axis=-1)


def _tblock(x, w, keep_t=None):
    g1, b1, qw, qb, pw, pb, g2, b2, f1w, f1b, f2w, f2b = w
    h = _ln(x, g1, b1)
    qkv = _mm(h, qw) + qb
    o = _attn(qkv, keep_t)
    x = x + _mm(o, pw) + pb
    h2 = _ln(x, g2, b2)
    ff = jax.nn.gelu(_mm(h2, f1w) + f1b)
    x = x + _mm(ff, f2w) + f2b
    return x


def _body(image_ref, conf_ref, *refs):
    nw = 7 + _NBLK * len(_BLOCK_KEYS)
    w = [r[...] for r in refs[:nw]]
    depth_ref, cg_ref = refs[nw], refs[nw + 1]
    st_s, dt_s = refs[nw + 2], refs[nw + 3]
    patch_w, patch_b = w[0], w[1]
    blocks = [w[2 + i * 12:2 + (i + 1) * 12] for i in range(_NBLK)]
    norm_g, norm_b, head_w, head_b, fill = w[2 + 12 * _NBLK:]

    conf = conf_ref[0, 0]
    rband = jnp.sum(conf.reshape(_G, _P, _W), axis=1)
    lane = jax.lax.broadcasted_iota(jnp.int32, (_W, _G), 0)
    col = jax.lax.broadcasted_iota(jnp.int32, (_W, _G), 1)
    blk_ones = (lane // _P == col).astype(jnp.float32)
    cg_grid = jnp.dot(rband, blk_ones, precision=jax.lax.Precision.HIGHEST,
                      preferred_element_type=jnp.float32) * (1.0 / (_P * _P))
    tok = jax.lax.broadcasted_iota(jnp.int32, (_N, _G), 0)
    j = jax.lax.broadcasted_iota(jnp.int32, (_N, _G), 1)
    oh_gy = (tok // _G == j).astype(jnp.float32)
    oh_gx = (tok % _G == j).astype(jnp.float32)
    per_gy = jnp.dot(oh_gy, cg_grid, precision=jax.lax.Precision.HIGHEST,
                     preferred_element_type=jnp.float32)
    cg = jnp.sum(per_gy * oh_gx, axis=-1, keepdims=True)
    cg_ref[0] = cg
    keep = cg < _THR
    keep_t = jax.lax.transpose(keep, (1, 0))

    bands = []
    for gy in range(_G):
        slab = image_ref[0, :, gy * _P:(gy + 1) * _P, :].reshape(
            3 * _P, _W)
        st_s[gy] = jax.lax.transpose(slab, (1, 0))
    for gy in range(_G):
        parts = [st_s[gy, px::_P, :] for px in range(_P)]
        bands.append(jnp.concatenate(parts, axis=1))
    patches = jnp.concatenate(bands, axis=0)
    x = _mm(patches, patch_w) + patch_b

    att = _tblock(x, blocks[0], keep_t=keep_t)
    x = jnp.where(keep, att, fill)

    for bw in blocks:
        x = _tblock(x, bw)

    x = _ln(x, norm_g, norm_b)
    dtok = _mm(x, head_w) + head_b

    for gy in range(_G):
        dg = dtok[gy * _G:(gy + 1) * _G, :]
        for px in range(_P):
            dt_s[gy, px::_P, :] = dg[:, px * _P:(px + 1) * _P]
    for gy in range(_G):
        band = jax.lax.transpose(dt_s[gy], (1, 0))
        depth_ref[0, 0, gy * _P:(gy + 1) * _P, :] = band


def _full_spec(shape):
    nd = len(shape)
    return pl.BlockSpec(shape, lambda b: (0,) * nd)


def kernel(image, confidence_map, sgm_depth_prior, params):
    del sgm_depth_prior

    def w2d(a):
        return a.reshape(1, -1) if a.ndim == 1 else a

    pw = params['patch_w'].reshape(_P, _P, 3, _D).transpose(
        1, 2, 0, 3).reshape(_P * _P * 3, _D)
    hw = params['head_w'].reshape(_D, _P, _P).transpose(
        0, 2, 1).reshape(_D, _P * _P)
    hb = params['head_b'].reshape(_P, _P).transpose(1, 0).reshape(1, _P * _P)

    weights = [pw, w2d(params['patch_b'])]
    for bp in params['blocks']:
        weights.extend(w2d(bp[k]) for k in _BLOCK_KEYS)
    weights.extend([w2d(params['norm_g']), w2d(params['norm_b']),
                    hw, hb, w2d(params['fill_token'])])

    in_specs = [
        pl.BlockSpec((1, 3, _H, _W), lambda b: (b, 0, 0, 0)),
        pl.BlockSpec((1, 1, _H, _W), lambda b: (b, 0, 0, 0)),
    ] + [_full_spec(wt.shape) for wt in weights]

    depth, cg = pl.pallas_call(
        _body,
        grid=(_B,),
        in_specs=in_specs,
        out_specs=[
            pl.BlockSpec((1, 1, _H, _W), lambda b: (b, 0, 0, 0)),
            pl.BlockSpec((1, _N, 1), lambda b: (b, 0, 0)),
        ],
        out_shape=[
            jax.ShapeDtypeStruct((_B, 1, _H, _W), jnp.float32),
            jax.ShapeDtypeStruct((_B, _N, 1), jnp.float32),
        ],
        scratch_shapes=[
            pltpu.VMEM((_G, _W, 48), jnp.float32),
            pltpu.VMEM((_G, _W, _P), jnp.float32),
        ],
        compiler_params=pltpu.CompilerParams(
            dimension_semantics=("parallel",)),
    )(image, confidence_map, *weights)

    cg_flat = cg.reshape(_B, _N)
    prune_ratio = jnp.mean((cg_flat >= _THR).astype(jnp.float32))
    return depth, prune_ratio, cg_flat.reshape(_B, _G, _G)


# speedup vs baseline: 2.2614x; 1.1070x over previous
import jax
import jax.numpy as jnp
import numpy as np
from jax.experimental import pallas as pl
from jax.experimental.pallas import tpu as pltpu

_B, _H, _W = 4, 512, 512
_P = 16
_G = 32
_N = _G * _G
_D = 192
_NH = 3
_DH = _D // _NH
_DFF = 4 * _D
_NBLK = 2
_THR = 0.5
_SCALE = 1.0 / np.sqrt(_DH)

_BLOCK_KEYS = ('ln1_g', 'ln1_b', 'qkv_w', 'qkv_b', 'proj_w', 'proj_b',
               'ln2_g', 'ln2_b', 'fc1_w', 'fc1_b', 'fc2_w', 'fc2_b')


def _mm(a, b):
    return jnp.dot(a.astype(jnp.bfloat16), b.astype(jnp.bfloat16),
                   preferred_element_type=jnp.float32)


def _ln(x, g, b):
    mu = jnp.mean(x, -1, keepdims=True)
    v = jnp.mean((x - mu) ** 2, -1, keepdims=True)
    return (x - mu) * jax.lax.rsqrt(v + 1e-6) * g + b


def _attn(qkv, keep_t):
    outs = []
    for h in range(_NH):
        q = qkv[:, h * _DH:(h + 1) * _DH]
        k = qkv[:, _D + h * _DH:_D + (h + 1) * _DH]
        v = qkv[:, 2 * _D + h * _DH:2 * _D + (h + 1) * _DH]
        logits = jax.lax.dot_general(
            q.astype(jnp.bfloat16), k.astype(jnp.bfloat16),
            (((1,), (1,)), ((), ())),
            preferred_element_type=jnp.float32) * _SCALE
        if keep_t is not None:
            logits = jnp.where(keep_t, logits, -1e30)
        e = jnp.exp(logits)
        s = jnp.sum(e, axis=-1, keepdims=True)
        outs.append(_mm(e, v) / s)
    return jnp.concatenate(outs, axis=-1)


def _tblock(x, w, keep_t=None):
    g1, b1, qw, qb, pw, pb, g2, b2, f1w, f1b, f2w, f2b = w
    h = _ln(x, g1, b1)
    qkv = _mm(h, qw) + qb
    o = _attn(qkv, keep_t)
    x = x + _mm(o, pw) + pb
    h2 = _ln(x, g2, b2)
    ff = jax.nn.gelu(_mm(h2, f1w) + f1b)
    x = x + _mm(ff, f2w) + f2b
    return x


def _body(image_ref, conf_ref, *refs):
    nw = 7 + _NBLK * len(_BLOCK_KEYS)
    w = [r[...] for r in refs[:nw]]
    depth_ref, cg_ref = refs[nw], refs[nw + 1]
    st_s, dt_s = refs[nw + 2], refs[nw + 3]
    patch_w, patch_b = w[0], w[1]
    blocks = [w[2 + i * 12:2 + (i + 1) * 12] for i in range(_NBLK)]
    norm_g, norm_b, head_w, head_b, fill = w[2 + 12 * _NBLK:]

    conf = conf_ref[0, 0]
    rband = jnp.sum(conf.reshape(_G, _P, _W), axis=1)
    lane = jax.lax.broadcasted_iota(jnp.int32, (_W, _G), 0)
    col = jax.lax.broadcasted_iota(jnp.int32, (_W, _G), 1)
    blk_ones = (lane // _P == col).astype(jnp.float32)
    cg_grid = jnp.dot(rband, blk_ones, precision=jax.lax.Precision.HIGHEST,
                      preferred_element_type=jnp.float32) * (1.0 / (_P * _P))
    tok = jax.lax.broadcasted_iota(jnp.int32, (_N, _G), 0)
    j = jax.lax.broadcasted_iota(jnp.int32, (_N, _G), 1)
    oh_gy = (tok // _G == j).astype(jnp.float32)
    oh_gx = (tok % _G == j).astype(jnp.float32)
    per_gy = jnp.dot(oh_gy, cg_grid, precision=jax.lax.Precision.HIGHEST,
                     preferred_element_type=jnp.float32)
    cg = jnp.sum(per_gy * oh_gx, axis=-1, keepdims=True)
    cg_ref[0] = cg
    keep = cg < _THR
    keep_t = jax.lax.transpose(keep, (1, 0))

    bands = []
    for gy in range(_G):
        slab = image_ref[0, :, gy * _P:(gy + 1) * _P, :].reshape(
            3 * _P, _W)
        st_s[gy] = jax.lax.transpose(slab, (1, 0))
    for gy in range(_G):
        parts = [st_s[gy, px::_P, :] for px in range(_P)]
        bands.append(jnp.concatenate(parts, axis=1))
    patches = jnp.concatenate(bands, axis=0)
    x = _mm(patches, patch_w) + patch_b

    att = _tblock(x, blocks[0], keep_t=keep_t)
    x = jnp.where(keep, att, fill)

    for bw in blocks:
        x = _tblock(x, bw)

    x = _ln(x, norm_g, norm_b)
    dtok = _mm(x, head_w) + head_b

    for gy in range(_G):
        dg = dtok[gy * _G:(gy + 1) * _G, :]
        for px in range(_P):
            dt_s[gy, px::_P, :] = dg[:, px * _P:(px + 1) * _P]
    for gy in range(_G):
        band = jax.lax.transpose(dt_s[gy], (1, 0))
        depth_ref[0, 0, gy * _P:(gy + 1) * _P, :] = band


def _full_spec(shape):
    nd = len(shape)
    return pl.BlockSpec(shape, lambda b: (0,) * nd)


def kernel(image, confidence_map, sgm_depth_prior, params):
    del sgm_depth_prior

    def w2d(a):
        return a.reshape(1, -1) if a.ndim == 1 else a

    pw = params['patch_w'].reshape(_P, _P, 3, _D).transpose(
        1, 2, 0, 3).reshape(_P * _P * 3, _D)
    hw = params['head_w'].reshape(_D, _P, _P).transpose(
        0, 2, 1).reshape(_D, _P * _P)
    hb = params['head_b'].reshape(_P, _P).transpose(1, 0).reshape(1, _P * _P)

    weights = [pw, w2d(params['patch_b'])]
    for bp in params['blocks']:
        weights.extend(w2d(bp[k]) for k in _BLOCK_KEYS)
    weights.extend([w2d(params['norm_g']), w2d(params['norm_b']),
                    hw, hb, w2d(params['fill_token'])])

    in_specs = [
        pl.BlockSpec((1, 3, _H, _W), lambda b: (b, 0, 0, 0)),
        pl.BlockSpec((1, 1, _H, _W), lambda b: (b, 0, 0, 0)),
    ] + [_full_spec(wt.shape) for wt in weights]

    depth, cg = pl.pallas_call(
        _body,
        grid=(_B,),
        in_specs=in_specs,
        out_specs=[
            pl.BlockSpec((1, 1, _H, _W), lambda b: (b, 0, 0, 0)),
            pl.BlockSpec((1, _N, 1), lambda b: (b, 0, 0)),
        ],
        out_shape=[
            jax.ShapeDtypeStruct((_B, 1, _H, _W), jnp.float32),
            jax.ShapeDtypeStruct((_B, _N, 1), jnp.float32),
        ],
        scratch_shapes=[
            pltpu.VMEM((_G, _W, 48), jnp.float32),
            pltpu.VMEM((_G, _W, _P), jnp.float32),
        ],
        compiler_params=pltpu.CompilerParams(
            dimension_semantics=("parallel",)),
    )(image, confidence_map, *weights)

    cg_flat = cg.reshape(_B, _N)
    prune_ratio = jnp.mean((cg_flat >= _THR).astype(jnp.float32))
    return depth, prune_ratio, cg_flat.reshape(_B, _G, _G)


# speedup vs baseline: 2.3140x; 1.0233x over previous
import jax
import jax.numpy as jnp
import numpy as np
from jax.experimental import pallas as pl
from jax.experimental.pallas import tpu as pltpu

_B, _H, _W = 4, 512, 512
_P = 16
_G = 32
_N = _G * _G
_D = 192
_NH = 3
_DH = _D // _NH
_DFF = 4 * _D
_NBLK = 2
_THR = 0.5
_SCALE = 1.0 / np.sqrt(_DH)

_BLOCK_KEYS = ('ln1_g', 'ln1_b', 'qkv_w', 'qkv_b', 'proj_w', 'proj_b',
               'ln2_g', 'ln2_b', 'fc1_w', 'fc1_b', 'fc2_w', 'fc2_b')


def _mm(a, b):
    return jnp.dot(a.astype(jnp.bfloat16), b.astype(jnp.bfloat16),
                   preferred_element_type=jnp.float32)


def _ln(x, g, b):
    mu = jnp.mean(x, -1, keepdims=True)
    v = jnp.mean(x * x, -1, keepdims=True) - mu * mu
    return (x - mu) * jax.lax.rsqrt(v + 1e-6) * g + b


def _attn(qkv, keep_t):
    outs = []
    for h in range(_NH):
        q = qkv[:, h * _DH:(h + 1) * _DH] * _SCALE
        k = qkv[:, _D + h * _DH:_D + (h + 1) * _DH]
        v = qkv[:, 2 * _D + h * _DH:2 * _D + (h + 1) * _DH]
        logits = jax.lax.dot_general(
            q.astype(jnp.bfloat16), k.astype(jnp.bfloat16),
            (((1,), (1,)), ((), ())),
            preferred_element_type=jnp.float32)
        if keep_t is not None:
            logits = jnp.where(keep_t, logits, -1e30)
        e = jnp.exp(logits)
        s = jnp.sum(e, axis=-1, keepdims=True)
        outs.append(_mm(e, v) / s)
    return jnp.concatenate(outs, axis=-1)


def _tblock(x, w, keep_t=None):
    g1, b1, qw, qb, pw, pb, g2, b2, f1w, f1b, f2w, f2b = w
    h = _ln(x, g1, b1)
    qkv = _mm(h, qw) + qb
    o = _attn(qkv, keep_t)
    x = x + _mm(o, pw) + pb
    h2 = _ln(x, g2, b2)
    ff = jax.nn.gelu(_mm(h2, f1w) + f1b)
    x = x + _mm(ff, f2w) + f2b
    return x


def _body(image_ref, conf_ref, *refs):
    nw = 7 + _NBLK * len(_BLOCK_KEYS)
    w = [r[...] for r in refs[:nw]]
    depth_ref, cg_ref = refs[nw], refs[nw + 1]
    st_s, dt_s = refs[nw + 2], refs[nw + 3]
    patch_w, patch_b = w[0], w[1]
    blocks = [w[2 + i * 12:2 + (i + 1) * 12] for i in range(_NBLK)]
    norm_g, norm_b, head_w, head_b, fill = w[2 + 12 * _NBLK:]

    conf = conf_ref[0, 0]
    rband = jnp.sum(conf.reshape(_G, _P, _W), axis=1)
    lane = jax.lax.broadcasted_iota(jnp.int32, (_W, _G), 0)
    col = jax.lax.broadcasted_iota(jnp.int32, (_W, _G), 1)
    blk_ones = (lane // _P == col).astype(jnp.float32)
    cg_grid = jnp.dot(rband, blk_ones, precision=jax.lax.Precision.HIGHEST,
                      preferred_element_type=jnp.float32) * (1.0 / (_P * _P))
    tok = jax.lax.broadcasted_iota(jnp.int32, (_N, _G), 0)
    j = jax.lax.broadcasted_iota(jnp.int32, (_N, _G), 1)
    oh_gy = (tok // _G == j).astype(jnp.float32)
    oh_gx = (tok % _G == j).astype(jnp.float32)
    per_gy = jnp.dot(oh_gy, cg_grid, precision=jax.lax.Precision.HIGHEST,
                     preferred_element_type=jnp.float32)
    cg = jnp.sum(per_gy * oh_gx, axis=-1, keepdims=True)
    cg_ref[0] = cg
    keep = cg < _THR
    keep_t = jax.lax.transpose(keep, (1, 0))

    bands = []
    for gy in range(_G):
        slab = image_ref[0, :, gy * _P:(gy + 1) * _P, :].reshape(
            3 * _P, _W)
        st_s[gy] = jax.lax.transpose(slab, (1, 0))
    for gy in range(_G):
        parts = [st_s[gy, px::_P, :] for px in range(_P)]
        bands.append(jnp.concatenate(parts, axis=1))
    patches = jnp.concatenate(bands, axis=0)
    x = _mm(patches, patch_w) + patch_b

    att = _tblock(x, blocks[0], keep_t=keep_t)
    x = jnp.where(keep, att, fill)

    for bw in blocks:
        x = _tblock(x, bw)

    x = _ln(x, norm_g, norm_b)
    dtok = _mm(x, head_w) + head_b

    for gy in range(_G):
        dg = dtok[gy * _G:(gy + 1) * _G, :]
        for px in range(_P):
            dt_s[gy, px::_P, :] = dg[:, px * _P:(px + 1) * _P]
    for gy in range(_G):
        band = jax.lax.transpose(dt_s[gy], (1, 0))
        depth_ref[0, 0, gy * _P:(gy + 1) * _P, :] = band


def _full_spec(shape):
    nd = len(shape)
    return pl.BlockSpec(shape, lambda b: (0,) * nd)


def kernel(image, confidence_map, sgm_depth_prior, params):
    del sgm_depth_prior

    def w2d(a):
        return a.reshape(1, -1) if a.ndim == 1 else a

    pw = params['patch_w'].reshape(_P, _P, 3, _D).transpose(
        1, 2, 0, 3).reshape(_P * _P * 3, _D)
    hw = params['head_w'].reshape(_D, _P, _P).transpose(
        0, 2, 1).reshape(_D, _P * _P)
    hb = params['head_b'].reshape(_P, _P).transpose(1, 0).reshape(1, _P * _P)

    weights = [pw, w2d(params['patch_b'])]
    for bp in params['blocks']:
        weights.extend(w2d(bp[k]) for k in _BLOCK_KEYS)
    weights.extend([w2d(params['norm_g']), w2d(params['norm_b']),
                    hw, hb, w2d(params['fill_token'])])

    in_specs = [
        pl.BlockSpec((1, 3, _H, _W), lambda b: (b, 0, 0, 0)),
        pl.BlockSpec((1, 1, _H, _W), lambda b: (b, 0, 0, 0)),
    ] + [_full_spec(wt.shape) for wt in weights]

    depth, cg = pl.pallas_call(
        _body,
        grid=(_B,),
        in_specs=in_specs,
        out_specs=[
            pl.BlockSpec((1, 1, _H, _W), lambda b: (b, 0, 0, 0)),
            pl.BlockSpec((1, _N, 1), lambda b: (b, 0, 0)),
        ],
        out_shape=[
            jax.ShapeDtypeStruct((_B, 1, _H, _W), jnp.float32),
            jax.ShapeDtypeStruct((_B, _N, 1), jnp.float32),
        ],
        scratch_shapes=[
            pltpu.VMEM((_G, _W, 48), jnp.float32),
            pltpu.VMEM((_G, _W, _P), jnp.float32),
        ],
        compiler_params=pltpu.CompilerParams(
            dimension_semantics=("parallel",)),
    )(image, confidence_map, *weights)

    cg_flat = cg.reshape(_B, _N)
    prune_ratio = jnp.mean((cg_flat >= _THR).astype(jnp.float32))
    return depth, prune_ratio, cg_flat.reshape(_B, _G, _G)


# speedup vs baseline: 2.3156x; 1.0007x over previous
import jax
import jax.numpy as jnp
import numpy as np
from jax.experimental import pallas as pl
from jax.experimental.pallas import tpu as pltpu

_B, _H, _W = 4, 512, 512
_P = 16
_G = 32
_N = _G * _G
_D = 192
_NH = 3
_DH = _D // _NH
_DFF = 4 * _D
_NBLK = 2
_THR = 0.5
_SCALE = 1.0 / np.sqrt(_DH)

_BLOCK_KEYS = ('ln1_g', 'ln1_b', 'qkv_w', 'qkv_b', 'proj_w', 'proj_b',
               'ln2_g', 'ln2_b', 'fc1_w', 'fc1_b', 'fc2_w', 'fc2_b')


def _mm(a, b):
    return jnp.dot(a.astype(jnp.bfloat16), b.astype(jnp.bfloat16),
                   preferred_element_type=jnp.float32)


def _ln(x, g, b):
    mu = jnp.mean(x, -1, keepdims=True)
    v = jnp.mean(x * x, -1, keepdims=True) - mu * mu
    return (x - mu) * jax.lax.rsqrt(v + 1e-6) * g + b


def _attn(qkv, keep_t):
    outs = []
    for h in range(_NH):
        q = qkv[:, h * _DH:(h + 1) * _DH] * jnp.bfloat16(_SCALE)
        k = qkv[:, _D + h * _DH:_D + (h + 1) * _DH]
        v = qkv[:, 2 * _D + h * _DH:2 * _D + (h + 1) * _DH]
        logits = jax.lax.dot_general(
            q, k, (((1,), (1,)), ((), ())),
            preferred_element_type=jnp.float32)
        if keep_t is not None:
            logits = jnp.where(keep_t, logits, -1e30)
        e = jnp.exp(logits)
        s = jnp.sum(e, axis=-1, keepdims=True)
        ev = jnp.dot(e.astype(jnp.bfloat16), v,
                     preferred_element_type=jnp.float32)
        outs.append(ev / s)
    return jnp.concatenate(outs, axis=-1)


def _tblock(x, w, keep_t=None):
    g1, b1, qw, qb, pw, pb, g2, b2, f1w, f1b, f2w, f2b = w
    h = _ln(x, g1, b1)
    qkv = (jnp.dot(h.astype(jnp.bfloat16), qw.astype(jnp.bfloat16),
                   preferred_element_type=jnp.float32)
           + qb).astype(jnp.bfloat16)
    o = _attn(qkv, keep_t)
    x = x + _mm(o, pw) + pb
    h2 = _ln(x, g2, b2)
    ff = jax.nn.gelu(_mm(h2, f1w) + f1b)
    x = x + _mm(ff, f2w) + f2b
    return x


def _body(image_ref, conf_ref, *refs):
    nw = 7 + _NBLK * len(_BLOCK_KEYS)
    w = [r[...] for r in refs[:nw]]
    depth_ref, cg_ref = refs[nw], refs[nw + 1]
    st_s, dt_s = refs[nw + 2], refs[nw + 3]
    patch_w, patch_b = w[0], w[1]
    blocks = [w[2 + i * 12:2 + (i + 1) * 12] for i in range(_NBLK)]
    norm_g, norm_b, head_w, head_b, fill = w[2 + 12 * _NBLK:]

    conf = conf_ref[0, 0]
    rband = jnp.sum(conf.reshape(_G, _P, _W), axis=1)
    lane = jax.lax.broadcasted_iota(jnp.int32, (_W, _G), 0)
    col = jax.lax.broadcasted_iota(jnp.int32, (_W, _G), 1)
    blk_ones = (lane // _P == col).astype(jnp.float32)
    cg_grid = jnp.dot(rband, blk_ones, precision=jax.lax.Precision.HIGHEST,
                      preferred_element_type=jnp.float32) * (1.0 / (_P * _P))
    tok = jax.lax.broadcasted_iota(jnp.int32, (_N, _G), 0)
    j = jax.lax.broadcasted_iota(jnp.int32, (_N, _G), 1)
    oh_gy = (tok // _G == j).astype(jnp.float32)
    oh_gx = (tok % _G == j).astype(jnp.float32)
    per_gy = jnp.dot(oh_gy, cg_grid, precision=jax.lax.Precision.HIGHEST,
                     preferred_element_type=jnp.float32)
    cg = jnp.sum(per_gy * oh_gx, axis=-1, keepdims=True)
    cg_ref[0] = cg
    keep = cg < _THR
    keep_t = jax.lax.transpose(keep, (1, 0))

    bands = []
    for gy in range(_G):
        slab = image_ref[0, :, gy * _P:(gy + 1) * _P, :].reshape(
            3 * _P, _W)
        st_s[gy] = jax.lax.transpose(slab, (1, 0))
    for gy in range(_G):
        parts = [st_s[gy, px::_P, :] for px in range(_P)]
        bands.append(jnp.concatenate(parts, axis=1))
    patches = jnp.concatenate(bands, axis=0)
    x = _mm(patches, patch_w) + patch_b

    att = _tblock(x, blocks[0], keep_t=keep_t)
    x = jnp.where(keep, att, fill)

    for bw in blocks:
        x = _tblock(x, bw)

    x = _ln(x, norm_g, norm_b)
    dtok = _mm(x, head_w) + head_b

    for gy in range(_G):
        dg = dtok[gy * _G:(gy + 1) * _G, :]
        for px in range(_P):
            dt_s[gy, px::_P, :] = dg[:, px * _P:(px + 1) * _P]
    for gy in range(_G):
        band = jax.lax.transpose(dt_s[gy], (1, 0))
        depth_ref[0, 0, gy * _P:(gy + 1) * _P, :] = band


def _full_spec(shape):
    nd = len(shape)
    return pl.BlockSpec(shape, lambda b: (0,) * nd)


def kernel(image, confidence_map, sgm_depth_prior, params):
    del sgm_depth_prior

    def w2d(a):
        return a.reshape(1, -1) if a.ndim == 1 else a

    pw = params['patch_w'].reshape(_P, _P, 3, _D).transpose(
        1, 2, 0, 3).reshape(_P * _P * 3, _D)
    hw = params['head_w'].reshape(_D, _P, _P).transpose(
        0, 2, 1).reshape(_D, _P * _P)
    hb = params['head_b'].reshape(_P, _P).transpose(1, 0).reshape(1, _P * _P)

    weights = [pw, w2d(params['patch_b'])]
    for bp in params['blocks']:
        weights.extend(w2d(bp[k]) for k in _BLOCK_KEYS)
    weights.extend([w2d(params['norm_g']), w2d(params['norm_b']),
                    hw, hb, w2d(params['fill_token'])])

    in_specs = [
        pl.BlockSpec((1, 3, _H, _W), lambda b: (b, 0, 0, 0)),
        pl.BlockSpec((1, 1, _H, _W), lambda b: (b, 0, 0, 0)),
    ] + [_full_spec(wt.shape) for wt in weights]

    depth, cg = pl.pallas_call(
        _body,
        grid=(_B,),
        in_specs=in_specs,
        out_specs=[
            pl.BlockSpec((1, 1, _H, _W), lambda b: (b, 0, 0, 0)),
            pl.BlockSpec((1, _N, 1), lambda b: (b, 0, 0)),
        ],
        out_shape=[
            jax.ShapeDtypeStruct((_B, 1, _H, _W), jnp.float32),
            jax.ShapeDtypeStruct((_B, _N, 1), jnp.float32),
        ],
        scratch_shapes=[
            pltpu.VMEM((_G, _W, 48), jnp.float32),
            pltpu.VMEM((_G, _W, _P), jnp.float32),
        ],
        compiler_params=pltpu.CompilerParams(
            dimension_semantics=("arbitrary",)),
    )(image, confidence_map, *weights)

    cg_flat = cg.reshape(_B, _N)
    prune_ratio = jnp.mean((cg_flat >= _THR).astype(jnp.float32))
    return depth, prune_ratio, cg_flat.reshape(_B, _G, _G)
